# Initial kernel scaffold; baseline (speedup 1.0000x reference)
import jax
import jax.numpy as jnp
from jax.experimental import pallas as pl


def kernel(node_features, edge_indices, edge_features, Wn, bn, We, be, Wm1, bm1, Wm2, bm2, Wu1, bu1, Wu2, bu2):
    raise NotImplementedError("write your pallas kernel here")


# speedup vs baseline: 2.6637x; 2.6637x over previous
import functools

import jax
import jax.numpy as jnp
from jax import lax
from jax.experimental import pallas as pl
from jax.experimental.pallas import tpu as pltpu
from jax.experimental.pallas import tpu_sc as plsc

N_NODES = 10000
N_EDGES = 320000
LANES = 128
N_ROWS = N_EDGES // LANES
NC, NS = 2, 16
NW = NC * NS
ROWS_PER_W = N_ROWS // NW
ROWS_EXTRA = N_ROWS % NW
NODES_PER_S = N_NODES // NS

_mesh = functools.partial(
    plsc.VectorSubcoreMesh, core_axis_name="c", subcore_axis_name="s")


def _worker_rows():
    c = lax.axis_index("c")
    s = lax.axis_index("s")
    wid = s * NC + c
    base = wid * ROWS_PER_W + jnp.minimum(wid, ROWS_EXTRA)
    n = ROWS_PER_W + (wid < ROWS_EXTRA).astype(jnp.int32)
    return c, s, base, n


@functools.partial(
    pl.kernel, mesh=_mesh(),
    out_type=jax.ShapeDtypeStruct((N_EDGES, 32), jnp.float32),
    compiler_params=pltpu.CompilerParams(use_tc_tiling_on_sc=False),
    scratch_types=[
        pltpu.VMEM((LANES,), jnp.int32),
        pltpu.VMEM((LANES, 32), jnp.float32),
        pltpu.SemaphoreType.DMA,
    ],
)
def _sc_gather(tab_hbm, src_hbm, out_hbm, idx_v, rows_v, sem):
    _, _, base, n = _worker_rows()

    def body(i, carry):
        r = base + i
        pltpu.sync_copy(src_hbm.at[r], idx_v)
        pltpu.async_copy(tab_hbm.at[idx_v], rows_v, sem).wait()
        pltpu.sync_copy(rows_v, out_hbm.at[pl.ds(r * LANES, LANES)])
        return carry

    lax.fori_loop(0, n, body, 0)


@functools.partial(
    pl.kernel, mesh=_mesh(),
    out_type=jax.ShapeDtypeStruct((NC, N_NODES, 32), jnp.float32),
    compiler_params=pltpu.CompilerParams(use_tc_tiling_on_sc=False),
    scratch_types=[
        pltpu.VMEM((ROWS_PER_W + 1, LANES), jnp.int32),
        pltpu.VMEM((LANES, 32), jnp.float32),
        pltpu.VMEM_SHARED((N_NODES, 32), jnp.float32),
        pltpu.SemaphoreType.DMA,
    ],
)
def _sc_scatter(msg_hbm, dst_hbm, zeros_hbm, out_hbm, idx_v, rows_v, acc, sem):
    c, s, base, n = _worker_rows()

    pltpu.sync_copy(zeros_hbm.at[pl.ds(s * NODES_PER_S, NODES_PER_S)],
                    acc.at[pl.ds(s * NODES_PER_S, NODES_PER_S)])
    plsc.subcore_barrier()

    pltpu.sync_copy(dst_hbm.at[pl.ds(base, ROWS_PER_W)],
                    idx_v.at[pl.ds(0, ROWS_PER_W)])

    @pl.when(n > ROWS_PER_W)
    def _():
        pltpu.sync_copy(dst_hbm.at[pl.ds(base + ROWS_PER_W, 1)],
                        idx_v.at[pl.ds(ROWS_PER_W, 1)])

    def body(j, carry):
        r = base + j
        pltpu.sync_copy(msg_hbm.at[pl.ds(r * LANES, LANES)], rows_v)
        pltpu.sync_copy(rows_v, acc.at[idx_v.at[j]], add=True)
        return carry

    lax.fori_loop(0, n, body, 0)
    plsc.subcore_barrier()

    pltpu.sync_copy(acc.at[pl.ds(s * NODES_PER_S, NODES_PER_S)],
                    out_hbm.at[c, pl.ds(s * NODES_PER_S, NODES_PER_S)])


def _node_body(nf_ref, w1_ref, wskip_ref, bskip_ref, msg_ref, skip_ref):
    nf = nf_ref[...]
    msg_ref[...] = jnp.dot(nf, w1_ref[...], preferred_element_type=jnp.float32)
    skip_ref[...] = (
        jnp.dot(nf, wskip_ref[...], preferred_element_type=jnp.float32)
        + bskip_ref[...])


def _edge_body(g_ref, ef_ref, we2_ref, bh_ref, wm2_ref, bm2_ref, msg_ref):
    h = (g_ref[...]
         + jnp.dot(ef_ref[...], we2_ref[...], preferred_element_type=jnp.float32)
         + bh_ref[...])
    msg_ref[...] = (
        jnp.dot(jnp.maximum(h, 0.0), wm2_ref[...],
                preferred_element_type=jnp.float32)
        + bm2_ref[...])


def _update_body(skip_ref, p_ref, wu1b_ref, wu2_ref, bu2_ref, out_ref):
    agg = p_ref[0] + p_ref[1]
    u = jnp.maximum(
        skip_ref[...]
        + jnp.dot(agg, wu1b_ref[...], preferred_element_type=jnp.float32),
        0.0)
    out_ref[...] = (
        jnp.dot(u, wu2_ref[...], preferred_element_type=jnp.float32)
        + bu2_ref[...])


def kernel(node_features, edge_indices, edge_features,
           Wn, bn, We, be, Wm1, bm1, Wm2, bm2, Wu1, bu1, Wu2, bu2):
    H = 32
    src = edge_indices[0].astype(jnp.int32).reshape(N_ROWS, LANES)
    dst = edge_indices[1].astype(jnp.int32).reshape(N_ROWS, LANES)

    W1 = Wn @ Wm1[:H]
    We2 = We @ Wm1[H:]
    b_h = (bn @ Wm1[:H] + be @ Wm1[H:] + bm1).reshape(1, H)
    bskip = bu1.reshape(1, H)
    bm2_r = bm2.reshape(1, H)
    bu2_r = bu2.reshape(1, H)

    node_msg, skip = pl.pallas_call(
        _node_body,
        out_shape=(
            jax.ShapeDtypeStruct((N_NODES, H), jnp.float32),
            jax.ShapeDtypeStruct((N_NODES, H), jnp.float32),
        ),
    )(node_features, W1, Wu1[:128], bskip)

    g = _sc_gather(node_msg, src)

    EB = 8000
    n_eb = N_EDGES // EB
    msg = pl.pallas_call(
        _edge_body,
        grid=(n_eb,),
        in_specs=[
            pl.BlockSpec((EB, H), lambda i: (i, 0)),
            pl.BlockSpec((EB, 16), lambda i: (i, 0)),
            pl.BlockSpec((16, H), lambda i: (0, 0)),
            pl.BlockSpec((1, H), lambda i: (0, 0)),
            pl.BlockSpec((H, H), lambda i: (0, 0)),
            pl.BlockSpec((1, H), lambda i: (0, 0)),
        ],
        out_specs=pl.BlockSpec((EB, H), lambda i: (i, 0)),
        out_shape=jax.ShapeDtypeStruct((N_EDGES, H), jnp.float32),
    )(g, edge_features, We2, b_h, Wm2, bm2_r)

    partials = _sc_scatter(msg, dst, jnp.zeros((N_NODES, H), jnp.float32))

    out = pl.pallas_call(
        _update_body,
        out_shape=jax.ShapeDtypeStruct((N_NODES, H), jnp.float32),
    )(skip, partials, Wu1[128:], Wu2, bu2_r)
    return out
